# baseline (speedup 1.0000x reference)
import functools

import jax
import jax.numpy as jnp
from jax import lax
from jax.experimental import pallas as pl
from jax.experimental.pallas import tpu as pltpu
from jax.experimental.pallas import tpu_sc as plsc

N_NODES = 10000
N_EDGES = 160000
D_FEAT = 256
D_EMB = 256
D_VG = 128
K = 16
N_TRAIN = 4096
V = 1.0
ALPHA = 0.2

_NPAD = 10240


def _proj_body(x_ref, W_ref, asrc_ref, adst_ref, hh_ref, a2_ref, m2_ref):
    i = pl.program_id(0)
    h = jnp.dot(x_ref[...], W_ref[...], preferred_element_type=jnp.float32)
    hh_ref[0] = h[:, :128]
    hh_ref[1] = h[:, 128:]
    a2 = jnp.concatenate(
        [h @ asrc_ref[...].reshape(D_EMB, 1), h @ adst_ref[...].reshape(D_EMB, 1)],
        axis=1,
    )
    a2_ref[...] = a2
    rows = i * 1024 + lax.broadcasted_iota(jnp.int32, (1024, 2), 0)
    a2m = jnp.where(rows < N_NODES, a2, -1e30)
    bmax = jnp.max(a2m, axis=0, keepdims=True)

    @pl.when(i == 0)
    def _():
        m2_ref[...] = bmax

    @pl.when(i > 0)
    def _():
        m2_ref[...] = jnp.maximum(m2_ref[...], bmax)


def _project(x, W, att_src, att_dst):
    xp = jnp.zeros((_NPAD, D_FEAT), jnp.float32).at[:N_NODES].set(x)
    grid = _NPAD // 1024
    hh, a2, m2 = pl.pallas_call(
        _proj_body,
        grid=(grid,),
        in_specs=[
            pl.BlockSpec((1024, D_FEAT), lambda i: (i, 0)),
            pl.BlockSpec((D_FEAT, D_EMB), lambda i: (0, 0)),
            pl.BlockSpec((D_EMB,), lambda i: (0,)),
            pl.BlockSpec((D_EMB,), lambda i: (0,)),
        ],
        out_specs=[
            pl.BlockSpec((2, 1024, 128), lambda i: (0, i, 0)),
            pl.BlockSpec((1024, 2), lambda i: (i, 0)),
            pl.BlockSpec((1, 2), lambda i: (0, 0)),
        ],
        out_shape=[
            jax.ShapeDtypeStruct((2, _NPAD, 128), jnp.float32),
            jax.ShapeDtypeStruct((_NPAD, 2), jnp.float32),
            jax.ShapeDtypeStruct((1, 2), jnp.float32),
        ],
    )(xp, W, att_src, att_dst)
    return hh, a2[:, 0], a2[:, 1], m2


_NTILES = 32
_EPT = N_EDGES // _NTILES
_EPT_PAD = ((_EPT + 15) // 16) * 16
_FULL_CHUNKS = _EPT // 16
_TAIL = _EPT - _FULL_CHUNKS * 16

_SC_MESH = plsc.VectorSubcoreMesh(core_axis_name="c", subcore_axis_name="s")
_SC_PARAMS = pltpu.CompilerParams(needs_layout_passes=False)


def _edge_logits_body(asrc_hbm, adst_hbm, src_hbm, dst_hbm, m_hbm,
                      ex_hbm, part_hbm,
                      asrc_v, adst_v, srcv, dstv, exv, denv, m_v):
    cid = lax.axis_index("c")
    sid = lax.axis_index("s")
    wid = sid * 2 + cid
    ebase = wid * _EPT

    pltpu.sync_copy(asrc_hbm, asrc_v)
    pltpu.sync_copy(adst_hbm, adst_v)
    pltpu.sync_copy(src_hbm.at[pl.ds(ebase, _EPT)], srcv.at[pl.ds(0, _EPT)])
    pltpu.sync_copy(dst_hbm.at[pl.ds(ebase, _EPT)], dstv.at[pl.ds(0, _EPT)])
    pltpu.sync_copy(m_hbm, m_v)
    m = m_v[...]

    zero16 = jnp.zeros((16,), jnp.float32)

    def _zero_body(i, carry):
        denv[pl.ds(i * 16, 16)] = zero16
        return carry

    lax.fori_loop(0, N_NODES // 16, _zero_body, 0)

    lanes = lax.iota(jnp.int32, 16)

    def _chunk(i, mask):
        off = i * 16
        sv = srcv[pl.ds(off, 16)]
        dv = dstv[pl.ds(off, 16)]
        if mask is not None:
            sv = jnp.where(mask, sv, 0)
            dv = jnp.where(mask, dv, 0)
        sa = plsc.load_gather(asrc_v, [sv])
        da = plsc.load_gather(adst_v, [dv])
        e = sa + da
        e = jnp.where(e >= 0.0, e, ALPHA * e)
        ex = jnp.exp(e - m)
        exv[pl.ds(off, 16)] = ex
        if mask is None:
            plsc.addupdate_scatter(denv, [dv], ex)
        else:
            plsc.addupdate_scatter(denv, [dv], ex, mask=mask)

    def _loop_body(i, carry):
        _chunk(i, None)
        return carry

    lax.fori_loop(0, _FULL_CHUNKS, _loop_body, 0)
    _chunk(_FULL_CHUNKS, lanes < _TAIL)

    pltpu.sync_copy(exv.at[pl.ds(0, _EPT)], ex_hbm.at[pl.ds(ebase, _EPT)])
    pltpu.sync_copy(denv, part_hbm.at[wid])


_edge_logits = pl.kernel(
    _edge_logits_body,
    out_type=[
        jax.ShapeDtypeStruct((N_EDGES,), jnp.float32),
        jax.ShapeDtypeStruct((_NTILES, N_NODES), jnp.float32),
    ],
    mesh=_SC_MESH,
    compiler_params=_SC_PARAMS,
    scratch_types=[
        pltpu.VMEM((N_NODES,), jnp.float32),
        pltpu.VMEM((N_NODES,), jnp.float32),
        pltpu.VMEM((_EPT_PAD,), jnp.int32),
        pltpu.VMEM((_EPT_PAD,), jnp.int32),
        pltpu.VMEM((_EPT_PAD,), jnp.float32),
        pltpu.VMEM((N_NODES,), jnp.float32),
        pltpu.VMEM((16,), jnp.float32),
    ],
)


_G = 80
_EPS = N_EDGES // 16
_NCH = _EPS // _G
_NCHP = 128
_ROWS_T = N_NODES // 16


_STRIPE = 632
_STRIPE_L = N_NODES - 15 * _STRIPE


def _agg_body(me_hbm, dst2_hbm, rden_hbm, hh_hbm,
              out_hbm,
              rden_v, dst2_v, mbA, mbB, attn_v, gb0, gb1, acc_sh,
              gsem0, gsem1, msemA, msemB, ssem0, ssem1):
    cid = lax.axis_index("c")
    sid = lax.axis_index("s")

    pltpu.sync_copy(rden_hbm, rden_v)
    pltpu.sync_copy(dst2_hbm.at[sid], dst2_v)

    zero16 = jnp.zeros((16,), jnp.float32)

    def _zrow(i, carry):
        for r in range(8):
            gb0[i, pl.ds(r * 16, 16)] = zero16
        return carry

    lax.fori_loop(0, _G, _zrow, 0)
    base = sid * _STRIPE

    @pl.when(sid < 15)
    def _():
        for k in range(7):
            pltpu.sync_copy(gb0, acc_sh.at[pl.ds(base + k * _G, _G)])
        pltpu.sync_copy(gb0.at[pl.ds(0, _STRIPE - 7 * _G)],
                        acc_sh.at[pl.ds(base + 7 * _G, _STRIPE - 7 * _G)])

    @pl.when(sid == 15)
    def _():
        for k in range(6):
            pltpu.sync_copy(gb0, acc_sh.at[pl.ds(base + k * _G, _G)])
        pltpu.sync_copy(gb0.at[pl.ds(0, _STRIPE_L - 6 * _G)],
                        acc_sh.at[pl.ds(base + 6 * _G, _STRIPE_L - 6 * _G)])

    plsc.subcore_barrier()

    def _launch(mb, p, gb, gsem):
        pltpu.async_copy(hh_hbm.at[cid].at[mb.at[p, 0]], gb, gsem)

    def _process(ci, mb, p, gb, gsem, ssem):
        for g in range(_G // 16):
            dv = dst2_v[ci, pl.ds(g * 16, 16)]
            rv = plsc.load_gather(rden_v, [dv])
            exv = plsc.bitcast(mb[p, 1, pl.ds(g * 16, 16)], jnp.float32)
            attn_v[pl.ds(g * 16, 16)] = exv * rv
        pltpu.make_async_copy(hh_hbm.at[cid].at[mb.at[p, 0]], gb, gsem).wait()

        def _egroup(g, c2):
            e0 = g * 16
            e0v = jnp.full((16,), 1, jnp.int32) * e0
            for j in range(16):
                av = plsc.load_gather(attn_v, [e0v + j])
                for r in range(8):
                    gb[e0 + j, pl.ds(r * 16, 16)] = (
                        gb[e0 + j, pl.ds(r * 16, 16)] * av)
            return c2

        lax.fori_loop(0, _G // 16, _egroup, 0)
        pltpu.async_copy(gb, acc_sh.at[dst2_v.at[ci]], add=True, sem=ssem)

    def _drain_scatter(gb, ssem):
        pltpu.make_async_copy(gb, acc_sh.at[dst2_v.at[0]], ssem).wait()

    pltpu.sync_copy(me_hbm.at[sid, pl.ds(0, 2)], mbA)
    pltpu.async_copy(me_hbm.at[sid, pl.ds(2, 2)], mbB, msemB)
    _launch(mbA, 0, gb0, gsem0)
    _launch(mbA, 1, gb1, gsem1)

    def _iter(k, carry):
        c0 = k * 4

        def _phase(cb, mb_cur, msem_cur, mb_nxt, msem_nxt):
            _process(cb, mb_cur, 0, gb0, gsem0, ssem0)

            @pl.when(cb <= _NCHP - 4)
            def _():
                pltpu.make_async_copy(me_hbm.at[sid, pl.ds(cb + 2, 2)],
                                      mb_nxt, msem_nxt).wait()
                _drain_scatter(gb0, ssem0)
                _launch(mb_nxt, 0, gb0, gsem0)

            _process(cb + 1, mb_cur, 1, gb1, gsem1, ssem1)

            @pl.when(cb <= _NCHP - 6)
            def _():
                pltpu.async_copy(me_hbm.at[sid, pl.ds(cb + 4, 2)],
                                 mb_cur, msem_cur)

            @pl.when(cb <= _NCHP - 4)
            def _():
                _drain_scatter(gb1, ssem1)
                _launch(mb_nxt, 1, gb1, gsem1)

        _phase(c0, mbA, msemA, mbB, msemB)
        _phase(c0 + 2, mbB, msemB, mbA, msemA)
        return carry

    lax.fori_loop(0, _NCHP // 4, _iter, 0)
    _drain_scatter(gb0, ssem0)
    _drain_scatter(gb1, ssem1)
    plsc.subcore_barrier()

    @pl.when(sid < 15)
    def _():
        pltpu.sync_copy(acc_sh.at[pl.ds(base, _STRIPE)],
                        out_hbm.at[cid, pl.ds(base, _STRIPE)])

    @pl.when(sid == 15)
    def _():
        pltpu.sync_copy(acc_sh.at[pl.ds(base, _STRIPE_L)],
                        out_hbm.at[cid, pl.ds(base, _STRIPE_L)])


_aggregate = pl.kernel(
    _agg_body,
    out_type=[
        jax.ShapeDtypeStruct((2, N_NODES, 128), jnp.float32),
    ],
    mesh=_SC_MESH,
    compiler_params=_SC_PARAMS,
    scratch_types=[
        pltpu.VMEM((N_NODES,), jnp.float32),
        pltpu.VMEM((_NCHP, _G), jnp.int32),
        pltpu.VMEM((2, 2, _G), jnp.int32),
        pltpu.VMEM((2, 2, _G), jnp.int32),
        pltpu.VMEM((_G,), jnp.float32),
        pltpu.VMEM((_G, 128), jnp.float32),
        pltpu.VMEM((_G, 128), jnp.float32),
        pltpu.VMEM_SHARED((N_NODES, 128), jnp.float32),
        pltpu.SemaphoreType.DMA,
        pltpu.SemaphoreType.DMA,
        pltpu.SemaphoreType.DMA,
        pltpu.SemaphoreType.DMA,
        pltpu.SemaphoreType.DMA,
        pltpu.SemaphoreType.DMA,
    ],
)


def _rden_body(part_ref, out_ref):
    s = jnp.sum(part_ref[...], axis=0, keepdims=True)
    out_ref[...] = 1.0 / (s + 1e-16)


_PPT = N_TRAIN // _NTILES


def _pair_gather_body(wc_hbm, emb_hbm, ew_hbm, ewec_hbm,
                      wc_v, ew_v, ec_v, sem):
    cid = lax.axis_index("c")
    sid = lax.axis_index("s")
    wid = sid * 2 + cid
    base = wid * _PPT

    pltpu.sync_copy(wc_hbm.at[wid], wc_v)
    pltpu.async_copy(emb_hbm.at[wc_v.at[0]], ew_v, sem).wait()
    pltpu.async_copy(emb_hbm.at[wc_v.at[1]], ec_v, sem).wait()

    def _mul(i, carry):
        for r in range(8):
            ec_v[i, pl.ds(r * 16, 16)] = (
                ec_v[i, pl.ds(r * 16, 16)] * ew_v[i, pl.ds(r * 16, 16)])
        return carry

    lax.fori_loop(0, _PPT, _mul, 0)
    pltpu.sync_copy(ew_v, ew_hbm.at[pl.ds(base, _PPT)])
    pltpu.sync_copy(ec_v, ewec_hbm.at[pl.ds(base, _PPT)])


_pair_gather = pl.kernel(
    _pair_gather_body,
    out_type=[
        jax.ShapeDtypeStruct((N_TRAIN, D_VG), jnp.float32),
        jax.ShapeDtypeStruct((N_TRAIN, D_VG), jnp.float32),
    ],
    mesh=_SC_MESH,
    compiler_params=_SC_PARAMS,
    scratch_types=[
        pltpu.VMEM((2, _PPT), jnp.int32),
        pltpu.VMEM((_PPT, D_VG), jnp.float32),
        pltpu.VMEM((_PPT, D_VG), jnp.float32),
        pltpu.SemaphoreType.DMA,
    ],
)


_RPT = N_TRAIN // 16


def _res_body(wc_hbm, qv_hbm, res_hbm,
              wc_v, qv_v, zbuf, acc_sh):
    cid = lax.axis_index("c")
    sid = lax.axis_index("s")

    pltpu.sync_copy(wc_hbm.at[sid], wc_v)

    zero16 = jnp.zeros((16,), jnp.float32)

    def _z(i, carry):
        for r in range(8):
            zbuf[i, pl.ds(r * 16, 16)] = zero16
        return carry

    lax.fori_loop(0, 104, _z, 0)
    base = sid * _STRIPE

    @pl.when(sid < 15)
    def _():
        for k in range(6):
            pltpu.sync_copy(zbuf, acc_sh.at[pl.ds(base + k * 104, 104)])
        pltpu.sync_copy(zbuf.at[pl.ds(0, 8)],
                        acc_sh.at[pl.ds(base + 624, 8)])

    @pl.when(sid == 15)
    def _():
        for k in range(5):
            pltpu.sync_copy(zbuf, acc_sh.at[pl.ds(base + k * 104, 104)])

    plsc.subcore_barrier()
    for k in range(_RPT // 128):
        pltpu.sync_copy(qv_hbm.at[sid, pl.ds(k * 128, 128)], qv_v)
        pltpu.sync_copy(qv_v,
                        acc_sh.at[wc_v.at[cid * (_RPT // 128) + k]],
                        add=True)
    plsc.subcore_barrier()

    @pl.when(sid < 15)
    def _():
        pltpu.sync_copy(acc_sh.at[pl.ds(base, _STRIPE)],
                        res_hbm.at[cid, pl.ds(base, _STRIPE)])

    @pl.when(sid == 15)
    def _():
        pltpu.sync_copy(acc_sh.at[pl.ds(base, _STRIPE_L)],
                        res_hbm.at[cid, pl.ds(base, _STRIPE_L)])


_res_scatter = pl.kernel(
    _res_body,
    out_type=[jax.ShapeDtypeStruct((2, N_NODES, 128), jnp.float32)],
    mesh=_SC_MESH,
    compiler_params=_SC_PARAMS,
    scratch_types=[
        pltpu.VMEM((4, 128), jnp.int32),
        pltpu.VMEM((128, 128), jnp.float32),
        pltpu.VMEM((104, 128), jnp.float32),
        pltpu.VMEM_SHARED((N_NODES, 128), jnp.float32),
    ],
)


def _vg_dense_body(ew_ref, ewec_ref, cw_ref, gum_ref, tmp_ref,
                   prior_ref, qvg_ref, nz_ref):
    cw = cw_ref[...]
    dn = (((1,), (1,)), ((), ()))
    pl_ = jax.lax.dot_general(ew_ref[...], cw, dn,
                              preferred_element_type=jnp.float32)
    ql = jax.lax.dot_general(ewec_ref[...], cw, dn,
                             preferred_element_type=jnp.float32)

    def _softmax(x):
        mx = jnp.max(x, axis=1, keepdims=True)
        e = jnp.exp(x - mx)
        return e / jnp.sum(e, axis=1, keepdims=True)

    prior_ref[...] = _softmax(pl_)
    qvg_ref[...] = _softmax(ql)
    zg = _softmax((ql + gum_ref[...]) / tmp_ref[0, 0])
    nz_ref[...] = jnp.dot(zg, cw, preferred_element_type=jnp.float32)


def _vg_dense(ew, ewec, comm_w, gumbel, temp):
    return pl.pallas_call(
        _vg_dense_body,
        grid=(4,),
        in_specs=[
            pl.BlockSpec((1024, D_VG), lambda i: (i, 0)),
            pl.BlockSpec((1024, D_VG), lambda i: (i, 0)),
            pl.BlockSpec((K, D_VG), lambda i: (0, 0)),
            pl.BlockSpec((1024, K), lambda i: (i, 0)),
            pl.BlockSpec((1, 1), lambda i: (0, 0)),
        ],
        out_specs=[
            pl.BlockSpec((1024, K), lambda i: (i, 0)),
            pl.BlockSpec((1024, K), lambda i: (i, 0)),
            pl.BlockSpec((1024, D_VG), lambda i: (i, 0)),
        ],
        out_shape=[
            jax.ShapeDtypeStruct((N_TRAIN, K), jnp.float32),
            jax.ShapeDtypeStruct((N_TRAIN, K), jnp.float32),
            jax.ShapeDtypeStruct((N_TRAIN, D_VG), jnp.float32),
        ],
    )(ew, ewec, comm_w, gumbel, temp.reshape(1, 1))


def _recon_body(nz_ref, ctx_ref, out_ref):
    out_ref[...] = jax.lax.dot_general(
        nz_ref[...], ctx_ref[...], (((1,), (1,)), ((), ())),
        preferred_element_type=jnp.float32)


def _recon(new_z, ctx_emb):
    return pl.pallas_call(
        _recon_body,
        grid=(4, 8),
        in_specs=[
            pl.BlockSpec((1024, D_VG), lambda i, j: (i, 0)),
            pl.BlockSpec((1280, D_VG), lambda i, j: (j, 0)),
        ],
        out_specs=pl.BlockSpec((1024, 1280), lambda i, j: (i, j)),
        out_shape=jax.ShapeDtypeStruct((N_TRAIN, N_NODES), jnp.float32),
    )(new_z, ctx_emb)


def _zq_body(out2_ref, cl_ref, res2_ref, z_ref, q_ref, qto_ref):
    o = jnp.concatenate([out2_ref[0], out2_ref[1]], axis=1)
    z = jnp.where(o > 0.0, o, jnp.exp(jnp.minimum(o, 0.0)) - 1.0)
    z_ref[...] = z
    cl = cl_ref[...]
    zc = jax.lax.dot_general(z, cl, (((1,), (1,)), ((), ())),
                             preferred_element_type=jnp.float32)
    z2 = jnp.sum(z * z, axis=1, keepdims=True)
    c2 = jnp.sum(cl * cl, axis=1)[None, :]
    d2 = z2 - 2.0 * zc + c2
    q = 1.0 / (1.0 + d2 / V)
    q = q / jnp.sum(q, axis=1, keepdims=True)
    q_ref[...] = q
    qto = q + 0.5 * (res2_ref[0, :, :K] + res2_ref[1, :, :K])
    qto_ref[...] = qto / jnp.sum(jnp.abs(qto), axis=1, keepdims=True)


def _zq(out2, cluster_layer, res2):
    return pl.pallas_call(
        _zq_body,
        grid=(10,),
        in_specs=[
            pl.BlockSpec((2, 1000, 128), lambda i: (0, i, 0)),
            pl.BlockSpec((K, D_EMB), lambda i: (0, 0)),
            pl.BlockSpec((2, 1000, 128), lambda i: (0, i, 0)),
        ],
        out_specs=[
            pl.BlockSpec((1000, D_EMB), lambda i: (i, 0)),
            pl.BlockSpec((1000, K), lambda i: (i, 0)),
            pl.BlockSpec((1000, K), lambda i: (i, 0)),
        ],
        out_shape=[
            jax.ShapeDtypeStruct((N_NODES, D_EMB), jnp.float32),
            jax.ShapeDtypeStruct((N_NODES, K), jnp.float32),
            jax.ShapeDtypeStruct((N_NODES, K), jnp.float32),
        ],
    )(out2, cluster_layer, res2)


def _reduce_rden(parts):
    return pl.pallas_call(
        _rden_body,
        grid=(5,),
        in_specs=[pl.BlockSpec((_NTILES, 2048), lambda i: (0, i))],
        out_specs=pl.BlockSpec((1, 2048), lambda i: (0, i)),
        out_shape=jax.ShapeDtypeStruct((1, N_NODES), jnp.float32),
    )(parts)[0]


def kernel(x, edge_index, w, c, temp, gumbel, W, att_src, att_dst,
           cluster_layer, node_emb, ctx_emb, comm_w):
    hh, a_src_p, a_dst_p, m2 = _project(x, W, att_src, att_dst)
    a_src = a_src_p[:N_NODES]
    a_dst = a_dst_p[:N_NODES]
    m = jax.nn.leaky_relu(m2[0, 0] + m2[0, 1], negative_slope=ALPHA)
    m16 = jnp.full((16,), m, jnp.float32)

    src = edge_index[0]
    dst = edge_index[1]
    ex, parts = _edge_logits(a_src, a_dst, src, dst, m16)
    rden = _reduce_rden(parts)
    pad3 = ((0, 0), (0, _NCHP - _NCH), (0, 0))
    me = jnp.stack(
        [jnp.pad(src.reshape(16, _NCH, _G), pad3),
         jnp.pad(lax.bitcast_convert_type(ex, jnp.int32).reshape(
             16, _NCH, _G), pad3)],
        axis=2,
    )
    dst3 = jnp.pad(dst.reshape(16, _NCH, _G), pad3)
    (out2,) = _aggregate(me, dst3, rden, hh)

    wc32 = jnp.stack([w.reshape(_NTILES, _PPT), c.reshape(_NTILES, _PPT)],
                     axis=1)
    ew, ewec = _pair_gather(wc32, node_emb)
    prior, q_vg, new_z = _vg_dense(ew, ewec, comm_w, gumbel, temp)
    recon_c = _recon(new_z, ctx_emb)

    wc16 = jnp.concatenate(
        [w.reshape(16, 2, 128), c.reshape(16, 2, 128)], axis=1)
    qvp = jnp.pad(q_vg, ((0, 0), (0, 128 - K)))
    (res2,) = _res_scatter(wc16, qvp.reshape(16, _RPT, 128))

    z, q, Q_to = _zq(out2, cluster_layer, res2)
    return (z, q, Q_to, prior, recon_c, q_vg, node_emb, comm_w)


# speedup vs baseline: 1.0206x; 1.0206x over previous
import functools

import jax
import jax.numpy as jnp
from jax import lax
from jax.experimental import pallas as pl
from jax.experimental.pallas import tpu as pltpu
from jax.experimental.pallas import tpu_sc as plsc

N_NODES = 10000
N_EDGES = 160000
D_FEAT = 256
D_EMB = 256
D_VG = 128
K = 16
N_TRAIN = 4096
V = 1.0
ALPHA = 0.2

_NPAD = 10240


def _proj_body(x_ref, W_ref, asrc_ref, adst_ref, hh_ref, a2_ref, m2_ref):
    i = pl.program_id(0)
    h = jnp.dot(x_ref[...], W_ref[...], preferred_element_type=jnp.float32)
    hh_ref[0] = h[:, :128]
    hh_ref[1] = h[:, 128:]
    a2 = jnp.concatenate(
        [h @ asrc_ref[...].reshape(D_EMB, 1), h @ adst_ref[...].reshape(D_EMB, 1)],
        axis=1,
    )
    a2_ref[...] = a2
    rows = i * 1024 + lax.broadcasted_iota(jnp.int32, (1024, 2), 0)
    a2m = jnp.where(rows < N_NODES, a2, -1e30)
    bmax = jnp.max(a2m, axis=0, keepdims=True)

    @pl.when(i == 0)
    def _():
        m2_ref[...] = bmax

    @pl.when(i > 0)
    def _():
        m2_ref[...] = jnp.maximum(m2_ref[...], bmax)


def _project(x, W, att_src, att_dst):
    xp = jnp.zeros((_NPAD, D_FEAT), jnp.float32).at[:N_NODES].set(x)
    grid = _NPAD // 1024
    hh, a2, m2 = pl.pallas_call(
        _proj_body,
        grid=(grid,),
        in_specs=[
            pl.BlockSpec((1024, D_FEAT), lambda i: (i, 0)),
            pl.BlockSpec((D_FEAT, D_EMB), lambda i: (0, 0)),
            pl.BlockSpec((D_EMB,), lambda i: (0,)),
            pl.BlockSpec((D_EMB,), lambda i: (0,)),
        ],
        out_specs=[
            pl.BlockSpec((2, 1024, 128), lambda i: (0, i, 0)),
            pl.BlockSpec((1024, 2), lambda i: (i, 0)),
            pl.BlockSpec((1, 2), lambda i: (0, 0)),
        ],
        out_shape=[
            jax.ShapeDtypeStruct((2, _NPAD, 128), jnp.float32),
            jax.ShapeDtypeStruct((_NPAD, 2), jnp.float32),
            jax.ShapeDtypeStruct((1, 2), jnp.float32),
        ],
    )(xp, W, att_src, att_dst)
    return hh, a2[:, 0], a2[:, 1], m2


_NTILES = 32
_EPT = N_EDGES // _NTILES
_EPT_PAD = ((_EPT + 15) // 16) * 16
_FULL_CHUNKS = _EPT // 16
_TAIL = _EPT - _FULL_CHUNKS * 16

_SC_MESH = plsc.VectorSubcoreMesh(core_axis_name="c", subcore_axis_name="s")
_SC_PARAMS = pltpu.CompilerParams(needs_layout_passes=False)


def _edge_logits_body(asrc_hbm, adst_hbm, src_hbm, dst_hbm, m_hbm,
                      ex_hbm, part_hbm,
                      asrc_v, adst_v, srcv, dstv, exv, denv, m_v):
    cid = lax.axis_index("c")
    sid = lax.axis_index("s")
    wid = sid * 2 + cid
    ebase = wid * _EPT

    pltpu.sync_copy(asrc_hbm, asrc_v)
    pltpu.sync_copy(adst_hbm, adst_v)
    pltpu.sync_copy(src_hbm.at[pl.ds(ebase, _EPT)], srcv.at[pl.ds(0, _EPT)])
    pltpu.sync_copy(dst_hbm.at[pl.ds(ebase, _EPT)], dstv.at[pl.ds(0, _EPT)])
    pltpu.sync_copy(m_hbm, m_v)
    m = m_v[...]

    zero16 = jnp.zeros((16,), jnp.float32)

    def _zero_body(i, carry):
        denv[pl.ds(i * 16, 16)] = zero16
        return carry

    lax.fori_loop(0, N_NODES // 16, _zero_body, 0)

    lanes = lax.iota(jnp.int32, 16)

    def _chunk(i, mask):
        off = i * 16
        sv = srcv[pl.ds(off, 16)]
        dv = dstv[pl.ds(off, 16)]
        if mask is not None:
            sv = jnp.where(mask, sv, 0)
            dv = jnp.where(mask, dv, 0)
        sa = plsc.load_gather(asrc_v, [sv])
        da = plsc.load_gather(adst_v, [dv])
        e = sa + da
        e = jnp.where(e >= 0.0, e, ALPHA * e)
        ex = jnp.exp(e - m)
        exv[pl.ds(off, 16)] = ex
        if mask is None:
            plsc.addupdate_scatter(denv, [dv], ex)
        else:
            plsc.addupdate_scatter(denv, [dv], ex, mask=mask)

    def _loop_body(i, carry):
        _chunk(i, None)
        return carry

    lax.fori_loop(0, _FULL_CHUNKS, _loop_body, 0)
    _chunk(_FULL_CHUNKS, lanes < _TAIL)

    pltpu.sync_copy(exv.at[pl.ds(0, _EPT)], ex_hbm.at[pl.ds(ebase, _EPT)])
    pltpu.sync_copy(denv, part_hbm.at[wid])


_edge_logits = pl.kernel(
    _edge_logits_body,
    out_type=[
        jax.ShapeDtypeStruct((N_EDGES,), jnp.float32),
        jax.ShapeDtypeStruct((_NTILES, N_NODES), jnp.float32),
    ],
    mesh=_SC_MESH,
    compiler_params=_SC_PARAMS,
    scratch_types=[
        pltpu.VMEM((N_NODES,), jnp.float32),
        pltpu.VMEM((N_NODES,), jnp.float32),
        pltpu.VMEM((_EPT_PAD,), jnp.int32),
        pltpu.VMEM((_EPT_PAD,), jnp.int32),
        pltpu.VMEM((_EPT_PAD,), jnp.float32),
        pltpu.VMEM((N_NODES,), jnp.float32),
        pltpu.VMEM((16,), jnp.float32),
    ],
)


_G = 80
_EPS = N_EDGES // 16
_NCH = _EPS // _G
_NCHP = 128
_ROWS_T = N_NODES // 16


_STRIPE = 632
_STRIPE_L = N_NODES - 15 * _STRIPE


def _agg_body(me_hbm, dst2_hbm, rden_hbm, hh_hbm,
              out_hbm,
              rden_v, dst2_v, mbA, attn_v, gb0, acc_sh, gsem0):
    cid = lax.axis_index("c")
    sid = lax.axis_index("s")

    pltpu.sync_copy(rden_hbm, rden_v)
    pltpu.sync_copy(dst2_hbm.at[sid], dst2_v)

    zero16 = jnp.zeros((16,), jnp.float32)

    def _zrow(i, carry):
        for r in range(8):
            gb0[i, pl.ds(r * 16, 16)] = zero16
        return carry

    lax.fori_loop(0, _G, _zrow, 0)
    base = sid * _STRIPE

    @pl.when(sid < 15)
    def _():
        for k in range(7):
            pltpu.sync_copy(gb0, acc_sh.at[pl.ds(base + k * _G, _G)])
        pltpu.sync_copy(gb0.at[pl.ds(0, _STRIPE - 7 * _G)],
                        acc_sh.at[pl.ds(base + 7 * _G, _STRIPE - 7 * _G)])

    @pl.when(sid == 15)
    def _():
        for k in range(6):
            pltpu.sync_copy(gb0, acc_sh.at[pl.ds(base + k * _G, _G)])
        pltpu.sync_copy(gb0.at[pl.ds(0, _STRIPE_L - 6 * _G)],
                        acc_sh.at[pl.ds(base + 6 * _G, _STRIPE_L - 6 * _G)])

    plsc.subcore_barrier()

    def _chunk(ci, carry):
        pltpu.sync_copy(me_hbm.at[sid, ci], mbA)
        pltpu.async_copy(hh_hbm.at[cid].at[mbA.at[0]], gb0, gsem0).wait()
        for g in range(_G // 16):
            dv = dst2_v[ci, pl.ds(g * 16, 16)]
            rv = plsc.load_gather(rden_v, [dv])
            exv = plsc.bitcast(mbA[1, pl.ds(g * 16, 16)], jnp.float32)
            attn_v[pl.ds(g * 16, 16)] = exv * rv

        def _egroup(g, c2):
            e0 = g * 16
            e0v = jnp.full((16,), 1, jnp.int32) * e0
            for j in range(16):
                av = plsc.load_gather(attn_v, [e0v + j])
                for r in range(8):
                    gb0[e0 + j, pl.ds(r * 16, 16)] = (
                        gb0[e0 + j, pl.ds(r * 16, 16)] * av)
            return c2

        lax.fori_loop(0, _G // 16, _egroup, 0)
        pltpu.sync_copy(gb0, acc_sh.at[dst2_v.at[ci]], add=True)
        return carry

    lax.fori_loop(0, _NCH, _chunk, 0)
    plsc.subcore_barrier()

    @pl.when(sid < 15)
    def _():
        pltpu.sync_copy(acc_sh.at[pl.ds(base, _STRIPE)],
                        out_hbm.at[cid, pl.ds(base, _STRIPE)])

    @pl.when(sid == 15)
    def _():
        pltpu.sync_copy(acc_sh.at[pl.ds(base, _STRIPE_L)],
                        out_hbm.at[cid, pl.ds(base, _STRIPE_L)])


_aggregate = pl.kernel(
    _agg_body,
    out_type=[
        jax.ShapeDtypeStruct((2, N_NODES, 128), jnp.float32),
    ],
    mesh=_SC_MESH,
    compiler_params=_SC_PARAMS,
    scratch_types=[
        pltpu.VMEM((N_NODES,), jnp.float32),
        pltpu.VMEM((_NCH, _G), jnp.int32),
        pltpu.VMEM((2, _G), jnp.int32),
        pltpu.VMEM((_G,), jnp.float32),
        pltpu.VMEM((_G, 128), jnp.float32),
        pltpu.VMEM_SHARED((N_NODES, 128), jnp.float32),
        pltpu.SemaphoreType.DMA,
    ],
)


def _rden_body(part_ref, out_ref):
    s = jnp.sum(part_ref[...], axis=0, keepdims=True)
    out_ref[...] = 1.0 / (s + 1e-16)


_PPT = N_TRAIN // _NTILES


def _pair_gather_body(wc_hbm, emb_hbm, ew_hbm, ewec_hbm,
                      wc_v, ew_v, ec_v, sem):
    cid = lax.axis_index("c")
    sid = lax.axis_index("s")
    wid = sid * 2 + cid
    base = wid * _PPT

    pltpu.sync_copy(wc_hbm.at[wid], wc_v)
    pltpu.async_copy(emb_hbm.at[wc_v.at[0]], ew_v, sem).wait()
    pltpu.async_copy(emb_hbm.at[wc_v.at[1]], ec_v, sem).wait()

    def _mul(i, carry):
        for r in range(8):
            ec_v[i, pl.ds(r * 16, 16)] = (
                ec_v[i, pl.ds(r * 16, 16)] * ew_v[i, pl.ds(r * 16, 16)])
        return carry

    lax.fori_loop(0, _PPT, _mul, 0)
    pltpu.sync_copy(ew_v, ew_hbm.at[pl.ds(base, _PPT)])
    pltpu.sync_copy(ec_v, ewec_hbm.at[pl.ds(base, _PPT)])


_pair_gather = pl.kernel(
    _pair_gather_body,
    out_type=[
        jax.ShapeDtypeStruct((N_TRAIN, D_VG), jnp.float32),
        jax.ShapeDtypeStruct((N_TRAIN, D_VG), jnp.float32),
    ],
    mesh=_SC_MESH,
    compiler_params=_SC_PARAMS,
    scratch_types=[
        pltpu.VMEM((2, _PPT), jnp.int32),
        pltpu.VMEM((_PPT, D_VG), jnp.float32),
        pltpu.VMEM((_PPT, D_VG), jnp.float32),
        pltpu.SemaphoreType.DMA,
    ],
)


_RPT = N_TRAIN // 16


def _res_body(wc_hbm, qv_hbm, res_hbm,
              wc_v, qv_v, zbuf, acc_sh):
    cid = lax.axis_index("c")
    sid = lax.axis_index("s")

    pltpu.sync_copy(wc_hbm.at[sid], wc_v)

    zero16 = jnp.zeros((16,), jnp.float32)

    def _z(i, carry):
        for r in range(8):
            zbuf[i, pl.ds(r * 16, 16)] = zero16
        return carry

    lax.fori_loop(0, 104, _z, 0)
    base = sid * _STRIPE

    @pl.when(sid < 15)
    def _():
        for k in range(6):
            pltpu.sync_copy(zbuf, acc_sh.at[pl.ds(base + k * 104, 104)])
        pltpu.sync_copy(zbuf.at[pl.ds(0, 8)],
                        acc_sh.at[pl.ds(base + 624, 8)])

    @pl.when(sid == 15)
    def _():
        for k in range(5):
            pltpu.sync_copy(zbuf, acc_sh.at[pl.ds(base + k * 104, 104)])

    plsc.subcore_barrier()
    for k in range(_RPT // 128):
        pltpu.sync_copy(qv_hbm.at[sid, pl.ds(k * 128, 128)], qv_v)
        pltpu.sync_copy(qv_v,
                        acc_sh.at[wc_v.at[cid * (_RPT // 128) + k]],
                        add=True)
    plsc.subcore_barrier()

    @pl.when(sid < 15)
    def _():
        pltpu.sync_copy(acc_sh.at[pl.ds(base, _STRIPE)],
                        res_hbm.at[cid, pl.ds(base, _STRIPE)])

    @pl.when(sid == 15)
    def _():
        pltpu.sync_copy(acc_sh.at[pl.ds(base, _STRIPE_L)],
                        res_hbm.at[cid, pl.ds(base, _STRIPE_L)])


_res_scatter = pl.kernel(
    _res_body,
    out_type=[jax.ShapeDtypeStruct((2, N_NODES, 128), jnp.float32)],
    mesh=_SC_MESH,
    compiler_params=_SC_PARAMS,
    scratch_types=[
        pltpu.VMEM((4, 128), jnp.int32),
        pltpu.VMEM((128, 128), jnp.float32),
        pltpu.VMEM((104, 128), jnp.float32),
        pltpu.VMEM_SHARED((N_NODES, 128), jnp.float32),
    ],
)


def _vg_dense_body(ew_ref, ewec_ref, cw_ref, gum_ref, tmp_ref,
                   prior_ref, qvg_ref, nz_ref):
    cw = cw_ref[...]
    dn = (((1,), (1,)), ((), ()))
    pl_ = jax.lax.dot_general(ew_ref[...], cw, dn,
                              preferred_element_type=jnp.float32)
    ql = jax.lax.dot_general(ewec_ref[...], cw, dn,
                             preferred_element_type=jnp.float32)

    def _softmax(x):
        mx = jnp.max(x, axis=1, keepdims=True)
        e = jnp.exp(x - mx)
        return e / jnp.sum(e, axis=1, keepdims=True)

    prior_ref[...] = _softmax(pl_)
    qvg_ref[...] = _softmax(ql)
    zg = _softmax((ql + gum_ref[...]) / tmp_ref[0, 0])
    nz_ref[...] = jnp.dot(zg, cw, preferred_element_type=jnp.float32)


def _vg_dense(ew, ewec, comm_w, gumbel, temp):
    return pl.pallas_call(
        _vg_dense_body,
        grid=(4,),
        in_specs=[
            pl.BlockSpec((1024, D_VG), lambda i: (i, 0)),
            pl.BlockSpec((1024, D_VG), lambda i: (i, 0)),
            pl.BlockSpec((K, D_VG), lambda i: (0, 0)),
            pl.BlockSpec((1024, K), lambda i: (i, 0)),
            pl.BlockSpec((1, 1), lambda i: (0, 0)),
        ],
        out_specs=[
            pl.BlockSpec((1024, K), lambda i: (i, 0)),
            pl.BlockSpec((1024, K), lambda i: (i, 0)),
            pl.BlockSpec((1024, D_VG), lambda i: (i, 0)),
        ],
        out_shape=[
            jax.ShapeDtypeStruct((N_TRAIN, K), jnp.float32),
            jax.ShapeDtypeStruct((N_TRAIN, K), jnp.float32),
            jax.ShapeDtypeStruct((N_TRAIN, D_VG), jnp.float32),
        ],
    )(ew, ewec, comm_w, gumbel, temp.reshape(1, 1))


def _recon_body(nz_ref, ctx_ref, out_ref):
    out_ref[...] = jax.lax.dot_general(
        nz_ref[...], ctx_ref[...], (((1,), (1,)), ((), ())),
        preferred_element_type=jnp.float32)


def _recon(new_z, ctx_emb):
    return pl.pallas_call(
        _recon_body,
        grid=(4, 8),
        in_specs=[
            pl.BlockSpec((1024, D_VG), lambda i, j: (i, 0)),
            pl.BlockSpec((1280, D_VG), lambda i, j: (j, 0)),
        ],
        out_specs=pl.BlockSpec((1024, 1280), lambda i, j: (i, j)),
        out_shape=jax.ShapeDtypeStruct((N_TRAIN, N_NODES), jnp.float32),
    )(new_z, ctx_emb)


def _zq_body(out2_ref, cl_ref, res2_ref, z_ref, q_ref, qto_ref):
    o = jnp.concatenate([out2_ref[0], out2_ref[1]], axis=1)
    z = jnp.where(o > 0.0, o, jnp.exp(jnp.minimum(o, 0.0)) - 1.0)
    z_ref[...] = z
    cl = cl_ref[...]
    zc = jax.lax.dot_general(z, cl, (((1,), (1,)), ((), ())),
                             preferred_element_type=jnp.float32)
    z2 = jnp.sum(z * z, axis=1, keepdims=True)
    c2 = jnp.sum(cl * cl, axis=1)[None, :]
    d2 = z2 - 2.0 * zc + c2
    q = 1.0 / (1.0 + d2 / V)
    q = q / jnp.sum(q, axis=1, keepdims=True)
    q_ref[...] = q
    qto = q + 0.5 * (res2_ref[0, :, :K] + res2_ref[1, :, :K])
    qto_ref[...] = qto / jnp.sum(jnp.abs(qto), axis=1, keepdims=True)


def _zq(out2, cluster_layer, res2):
    return pl.pallas_call(
        _zq_body,
        grid=(10,),
        in_specs=[
            pl.BlockSpec((2, 1000, 128), lambda i: (0, i, 0)),
            pl.BlockSpec((K, D_EMB), lambda i: (0, 0)),
            pl.BlockSpec((2, 1000, 128), lambda i: (0, i, 0)),
        ],
        out_specs=[
            pl.BlockSpec((1000, D_EMB), lambda i: (i, 0)),
            pl.BlockSpec((1000, K), lambda i: (i, 0)),
            pl.BlockSpec((1000, K), lambda i: (i, 0)),
        ],
        out_shape=[
            jax.ShapeDtypeStruct((N_NODES, D_EMB), jnp.float32),
            jax.ShapeDtypeStruct((N_NODES, K), jnp.float32),
            jax.ShapeDtypeStruct((N_NODES, K), jnp.float32),
        ],
    )(out2, cluster_layer, res2)


def _reduce_rden(parts):
    return pl.pallas_call(
        _rden_body,
        grid=(5,),
        in_specs=[pl.BlockSpec((_NTILES, 2048), lambda i: (0, i))],
        out_specs=pl.BlockSpec((1, 2048), lambda i: (0, i)),
        out_shape=jax.ShapeDtypeStruct((1, N_NODES), jnp.float32),
    )(parts)[0]


def kernel(x, edge_index, w, c, temp, gumbel, W, att_src, att_dst,
           cluster_layer, node_emb, ctx_emb, comm_w):
    hh, a_src_p, a_dst_p, m2 = _project(x, W, att_src, att_dst)
    a_src = a_src_p[:N_NODES]
    a_dst = a_dst_p[:N_NODES]
    m = jax.nn.leaky_relu(m2[0, 0] + m2[0, 1], negative_slope=ALPHA)
    m16 = jnp.full((16,), m, jnp.float32)

    src = edge_index[0]
    dst = edge_index[1]
    ex, parts = _edge_logits(a_src, a_dst, src, dst, m16)
    rden = _reduce_rden(parts)
    me = jnp.stack(
        [src.reshape(16, _NCH, _G),
         lax.bitcast_convert_type(ex, jnp.int32).reshape(16, _NCH, _G)],
        axis=2,
    )
    (out2,) = _aggregate(me, dst.reshape(16, _NCH, _G), rden, hh)

    wc32 = jnp.stack([w.reshape(_NTILES, _PPT), c.reshape(_NTILES, _PPT)],
                     axis=1)
    ew, ewec = _pair_gather(wc32, node_emb)
    prior, q_vg, new_z = _vg_dense(ew, ewec, comm_w, gumbel, temp)
    recon_c = _recon(new_z, ctx_emb)

    wc16 = jnp.concatenate(
        [w.reshape(16, 2, 128), c.reshape(16, 2, 128)], axis=1)
    qvp = jnp.pad(q_vg, ((0, 0), (0, 128 - K)))
    (res2,) = _res_scatter(wc16, qvp.reshape(16, _RPT, 128))

    z, q, Q_to = _zq(out2, cluster_layer, res2)
    return (z, q, Q_to, prior, recon_c, q_vg, node_emb, comm_w)


# speedup vs baseline: 1.0850x; 1.0631x over previous
import functools

import jax
import jax.numpy as jnp
from jax import lax
from jax.experimental import pallas as pl
from jax.experimental.pallas import tpu as pltpu
from jax.experimental.pallas import tpu_sc as plsc

N_NODES = 10000
N_EDGES = 160000
D_FEAT = 256
D_EMB = 256
D_VG = 128
K = 16
N_TRAIN = 4096
V = 1.0
ALPHA = 0.2

_NPAD = 10240


def _proj_body(x_ref, W_ref, asrc_ref, adst_ref, hh_ref, a2_ref, m2_ref):
    i = pl.program_id(0)
    h = jnp.dot(x_ref[...], W_ref[...], preferred_element_type=jnp.float32)
    hh_ref[0] = h[:, :128]
    hh_ref[1] = h[:, 128:]
    a2 = jnp.concatenate(
        [h @ asrc_ref[...].reshape(D_EMB, 1), h @ adst_ref[...].reshape(D_EMB, 1)],
        axis=1,
    )
    a2_ref[...] = a2
    rows = i * 1024 + lax.broadcasted_iota(jnp.int32, (1024, 2), 0)
    a2m = jnp.where(rows < N_NODES, a2, -1e30)
    bmax = jnp.max(a2m, axis=0, keepdims=True)

    @pl.when(i == 0)
    def _():
        m2_ref[...] = bmax

    @pl.when(i > 0)
    def _():
        m2_ref[...] = jnp.maximum(m2_ref[...], bmax)


def _project(x, W, att_src, att_dst):
    xp = jnp.zeros((_NPAD, D_FEAT), jnp.float32).at[:N_NODES].set(x)
    grid = _NPAD // 1024
    hh, a2, m2 = pl.pallas_call(
        _proj_body,
        grid=(grid,),
        in_specs=[
            pl.BlockSpec((1024, D_FEAT), lambda i: (i, 0)),
            pl.BlockSpec((D_FEAT, D_EMB), lambda i: (0, 0)),
            pl.BlockSpec((D_EMB,), lambda i: (0,)),
            pl.BlockSpec((D_EMB,), lambda i: (0,)),
        ],
        out_specs=[
            pl.BlockSpec((2, 1024, 128), lambda i: (0, i, 0)),
            pl.BlockSpec((1024, 2), lambda i: (i, 0)),
            pl.BlockSpec((1, 2), lambda i: (0, 0)),
        ],
        out_shape=[
            jax.ShapeDtypeStruct((2, _NPAD, 128), jnp.float32),
            jax.ShapeDtypeStruct((_NPAD, 2), jnp.float32),
            jax.ShapeDtypeStruct((1, 2), jnp.float32),
        ],
    )(xp, W, att_src, att_dst)
    return hh, a2[:, 0], a2[:, 1], m2


_NTILES = 32
_EPT = N_EDGES // _NTILES
_EPT_PAD = ((_EPT + 15) // 16) * 16
_FULL_CHUNKS = _EPT // 16
_TAIL = _EPT - _FULL_CHUNKS * 16

_SC_MESH = plsc.VectorSubcoreMesh(core_axis_name="c", subcore_axis_name="s")
_SC_PARAMS = pltpu.CompilerParams(needs_layout_passes=False)


def _edge_logits_body(asrc_hbm, adst_hbm, src_hbm, dst_hbm, m_hbm,
                      ex_hbm, part_hbm,
                      asrc_v, adst_v, srcv, dstv, exv, denv, m_v):
    cid = lax.axis_index("c")
    sid = lax.axis_index("s")
    wid = sid * 2 + cid
    ebase = wid * _EPT

    pltpu.sync_copy(asrc_hbm, asrc_v)
    pltpu.sync_copy(adst_hbm, adst_v)
    pltpu.sync_copy(src_hbm.at[pl.ds(ebase, _EPT)], srcv.at[pl.ds(0, _EPT)])
    pltpu.sync_copy(dst_hbm.at[pl.ds(ebase, _EPT)], dstv.at[pl.ds(0, _EPT)])
    pltpu.sync_copy(m_hbm, m_v)
    m = m_v[...]

    zero16 = jnp.zeros((16,), jnp.float32)

    def _zero_body(i, carry):
        denv[pl.ds(i * 16, 16)] = zero16
        return carry

    lax.fori_loop(0, N_NODES // 16, _zero_body, 0)

    lanes = lax.iota(jnp.int32, 16)

    def _chunk(i, mask):
        off = i * 16
        sv = srcv[pl.ds(off, 16)]
        dv = dstv[pl.ds(off, 16)]
        if mask is not None:
            sv = jnp.where(mask, sv, 0)
            dv = jnp.where(mask, dv, 0)
        sa = plsc.load_gather(asrc_v, [sv])
        da = plsc.load_gather(adst_v, [dv])
        e = sa + da
        e = jnp.where(e >= 0.0, e, ALPHA * e)
        ex = jnp.exp(e - m)
        exv[pl.ds(off, 16)] = ex
        if mask is None:
            plsc.addupdate_scatter(denv, [dv], ex)
        else:
            plsc.addupdate_scatter(denv, [dv], ex, mask=mask)

    def _loop_body(i, carry):
        _chunk(i, None)
        return carry

    lax.fori_loop(0, _FULL_CHUNKS, _loop_body, 0)
    _chunk(_FULL_CHUNKS, lanes < _TAIL)

    pltpu.sync_copy(exv.at[pl.ds(0, _EPT)], ex_hbm.at[pl.ds(ebase, _EPT)])
    pltpu.sync_copy(denv, part_hbm.at[wid])


_edge_logits = pl.kernel(
    _edge_logits_body,
    out_type=[
        jax.ShapeDtypeStruct((N_EDGES,), jnp.float32),
        jax.ShapeDtypeStruct((_NTILES, N_NODES), jnp.float32),
    ],
    mesh=_SC_MESH,
    compiler_params=_SC_PARAMS,
    scratch_types=[
        pltpu.VMEM((N_NODES,), jnp.float32),
        pltpu.VMEM((N_NODES,), jnp.float32),
        pltpu.VMEM((_EPT_PAD,), jnp.int32),
        pltpu.VMEM((_EPT_PAD,), jnp.int32),
        pltpu.VMEM((_EPT_PAD,), jnp.float32),
        pltpu.VMEM((N_NODES,), jnp.float32),
        pltpu.VMEM((16,), jnp.float32),
    ],
)


_G = 80
_EPS = N_EDGES // 16
_NCH = _EPS // _G
_NCHP = 128
_ROWS_T = N_NODES // 16


_STRIPE = 632
_STRIPE_L = N_NODES - 15 * _STRIPE


def _agg_body(me_hbm, dst2_hbm, rden_hbm, hh_hbm,
              out_hbm,
              rden_v, dst2_v, mbA, attn_v, gb0, acc_sh, gsem0):
    cid = lax.axis_index("c")
    sid = lax.axis_index("s")

    pltpu.sync_copy(rden_hbm, rden_v)
    pltpu.sync_copy(dst2_hbm.at[sid], dst2_v)

    zero16 = jnp.zeros((16,), jnp.float32)

    def _zrow(i, carry):
        for r in range(8):
            gb0[i, pl.ds(r * 16, 16)] = zero16
        return carry

    lax.fori_loop(0, _G, _zrow, 0)
    base = sid * _STRIPE

    @pl.when(sid < 15)
    def _():
        for k in range(7):
            pltpu.sync_copy(gb0, acc_sh.at[pl.ds(base + k * _G, _G)])
        pltpu.sync_copy(gb0.at[pl.ds(0, _STRIPE - 7 * _G)],
                        acc_sh.at[pl.ds(base + 7 * _G, _STRIPE - 7 * _G)])

    @pl.when(sid == 15)
    def _():
        for k in range(6):
            pltpu.sync_copy(gb0, acc_sh.at[pl.ds(base + k * _G, _G)])
        pltpu.sync_copy(gb0.at[pl.ds(0, _STRIPE_L - 6 * _G)],
                        acc_sh.at[pl.ds(base + 6 * _G, _STRIPE_L - 6 * _G)])

    plsc.subcore_barrier()

    def _chunk(ci, carry):
        pltpu.sync_copy(me_hbm.at[sid, ci], mbA)
        pltpu.async_copy(hh_hbm.at[cid].at[mbA.at[0]], gb0, gsem0).wait()
        for g in range(_G // 16):
            dv = dst2_v[ci, pl.ds(g * 16, 16)]
            rv = plsc.load_gather(rden_v, [dv])
            exv = plsc.bitcast(mbA[1, pl.ds(g * 16, 16)], jnp.float32)
            attn_v[pl.ds(g * 16, 16)] = exv * rv

        def _egroup(g, c2):
            avec = attn_v[pl.ds(g * 16, 16)]
            e0 = g * 16
            for j in range(16):
                av = jnp.broadcast_to(avec[j], (16,))
                for r in range(8):
                    gb0[e0 + j, pl.ds(r * 16, 16)] = (
                        gb0[e0 + j, pl.ds(r * 16, 16)] * av)
            return c2

        lax.fori_loop(0, _G // 16, _egroup, 0)
        pltpu.sync_copy(gb0, acc_sh.at[dst2_v.at[ci]], add=True)
        return carry

    lax.fori_loop(0, _NCH, _chunk, 0)
    plsc.subcore_barrier()

    @pl.when(sid < 15)
    def _():
        pltpu.sync_copy(acc_sh.at[pl.ds(base, _STRIPE)],
                        out_hbm.at[cid, pl.ds(base, _STRIPE)])

    @pl.when(sid == 15)
    def _():
        pltpu.sync_copy(acc_sh.at[pl.ds(base, _STRIPE_L)],
                        out_hbm.at[cid, pl.ds(base, _STRIPE_L)])


_aggregate = pl.kernel(
    _agg_body,
    out_type=[
        jax.ShapeDtypeStruct((2, N_NODES, 128), jnp.float32),
    ],
    mesh=_SC_MESH,
    compiler_params=_SC_PARAMS,
    scratch_types=[
        pltpu.VMEM((N_NODES,), jnp.float32),
        pltpu.VMEM((_NCH, _G), jnp.int32),
        pltpu.VMEM((2, _G), jnp.int32),
        pltpu.VMEM((_G,), jnp.float32),
        pltpu.VMEM((_G, 128), jnp.float32),
        pltpu.VMEM_SHARED((N_NODES, 128), jnp.float32),
        pltpu.SemaphoreType.DMA,
    ],
)


def _rden_body(part_ref, out_ref):
    s = jnp.sum(part_ref[...], axis=0, keepdims=True)
    out_ref[...] = 1.0 / (s + 1e-16)


_PPT = N_TRAIN // _NTILES


def _pair_gather_body(wc_hbm, emb_hbm, ew_hbm, ewec_hbm,
                      wc_v, ew_v, ec_v, sem):
    cid = lax.axis_index("c")
    sid = lax.axis_index("s")
    wid = sid * 2 + cid
    base = wid * _PPT

    pltpu.sync_copy(wc_hbm.at[wid], wc_v)
    pltpu.async_copy(emb_hbm.at[wc_v.at[0]], ew_v, sem).wait()
    pltpu.async_copy(emb_hbm.at[wc_v.at[1]], ec_v, sem).wait()

    def _mul(i, carry):
        for r in range(8):
            ec_v[i, pl.ds(r * 16, 16)] = (
                ec_v[i, pl.ds(r * 16, 16)] * ew_v[i, pl.ds(r * 16, 16)])
        return carry

    lax.fori_loop(0, _PPT, _mul, 0)
    pltpu.sync_copy(ew_v, ew_hbm.at[pl.ds(base, _PPT)])
    pltpu.sync_copy(ec_v, ewec_hbm.at[pl.ds(base, _PPT)])


_pair_gather = pl.kernel(
    _pair_gather_body,
    out_type=[
        jax.ShapeDtypeStruct((N_TRAIN, D_VG), jnp.float32),
        jax.ShapeDtypeStruct((N_TRAIN, D_VG), jnp.float32),
    ],
    mesh=_SC_MESH,
    compiler_params=_SC_PARAMS,
    scratch_types=[
        pltpu.VMEM((2, _PPT), jnp.int32),
        pltpu.VMEM((_PPT, D_VG), jnp.float32),
        pltpu.VMEM((_PPT, D_VG), jnp.float32),
        pltpu.SemaphoreType.DMA,
    ],
)


_RPT = N_TRAIN // 16


def _res_body(wc_hbm, qv_hbm, res_hbm,
              wc_v, qv_v, zbuf, acc_sh):
    cid = lax.axis_index("c")
    sid = lax.axis_index("s")

    pltpu.sync_copy(wc_hbm.at[sid], wc_v)

    zero16 = jnp.zeros((16,), jnp.float32)

    def _z(i, carry):
        for r in range(8):
            zbuf[i, pl.ds(r * 16, 16)] = zero16
        return carry

    lax.fori_loop(0, 104, _z, 0)
    base = sid * _STRIPE

    @pl.when(sid < 15)
    def _():
        for k in range(6):
            pltpu.sync_copy(zbuf, acc_sh.at[pl.ds(base + k * 104, 104)])
        pltpu.sync_copy(zbuf.at[pl.ds(0, 8)],
                        acc_sh.at[pl.ds(base + 624, 8)])

    @pl.when(sid == 15)
    def _():
        for k in range(5):
            pltpu.sync_copy(zbuf, acc_sh.at[pl.ds(base + k * 104, 104)])

    plsc.subcore_barrier()
    for k in range(_RPT // 128):
        pltpu.sync_copy(qv_hbm.at[sid, pl.ds(k * 128, 128)], qv_v)
        pltpu.sync_copy(qv_v,
                        acc_sh.at[wc_v.at[cid * (_RPT // 128) + k]],
                        add=True)
    plsc.subcore_barrier()

    @pl.when(sid < 15)
    def _():
        pltpu.sync_copy(acc_sh.at[pl.ds(base, _STRIPE)],
                        res_hbm.at[cid, pl.ds(base, _STRIPE)])

    @pl.when(sid == 15)
    def _():
        pltpu.sync_copy(acc_sh.at[pl.ds(base, _STRIPE_L)],
                        res_hbm.at[cid, pl.ds(base, _STRIPE_L)])


_res_scatter = pl.kernel(
    _res_body,
    out_type=[jax.ShapeDtypeStruct((2, N_NODES, 128), jnp.float32)],
    mesh=_SC_MESH,
    compiler_params=_SC_PARAMS,
    scratch_types=[
        pltpu.VMEM((4, 128), jnp.int32),
        pltpu.VMEM((128, 128), jnp.float32),
        pltpu.VMEM((104, 128), jnp.float32),
        pltpu.VMEM_SHARED((N_NODES, 128), jnp.float32),
    ],
)


def _vg_dense_body(ew_ref, ewec_ref, cw_ref, gum_ref, tmp_ref,
                   prior_ref, qvg_ref, nz_ref):
    cw = cw_ref[...]
    dn = (((1,), (1,)), ((), ()))
    pl_ = jax.lax.dot_general(ew_ref[...], cw, dn,
                              preferred_element_type=jnp.float32)
    ql = jax.lax.dot_general(ewec_ref[...], cw, dn,
                             preferred_element_type=jnp.float32)

    def _softmax(x):
        mx = jnp.max(x, axis=1, keepdims=True)
        e = jnp.exp(x - mx)
        return e / jnp.sum(e, axis=1, keepdims=True)

    prior_ref[...] = _softmax(pl_)
    qvg_ref[...] = _softmax(ql)
    zg = _softmax((ql + gum_ref[...]) / tmp_ref[0, 0])
    nz_ref[...] = jnp.dot(zg, cw, preferred_element_type=jnp.float32)


def _vg_dense(ew, ewec, comm_w, gumbel, temp):
    return pl.pallas_call(
        _vg_dense_body,
        grid=(4,),
        in_specs=[
            pl.BlockSpec((1024, D_VG), lambda i: (i, 0)),
            pl.BlockSpec((1024, D_VG), lambda i: (i, 0)),
            pl.BlockSpec((K, D_VG), lambda i: (0, 0)),
            pl.BlockSpec((1024, K), lambda i: (i, 0)),
            pl.BlockSpec((1, 1), lambda i: (0, 0)),
        ],
        out_specs=[
            pl.BlockSpec((1024, K), lambda i: (i, 0)),
            pl.BlockSpec((1024, K), lambda i: (i, 0)),
            pl.BlockSpec((1024, D_VG), lambda i: (i, 0)),
        ],
        out_shape=[
            jax.ShapeDtypeStruct((N_TRAIN, K), jnp.float32),
            jax.ShapeDtypeStruct((N_TRAIN, K), jnp.float32),
            jax.ShapeDtypeStruct((N_TRAIN, D_VG), jnp.float32),
        ],
    )(ew, ewec, comm_w, gumbel, temp.reshape(1, 1))


def _recon_body(nz_ref, ctx_ref, out_ref):
    out_ref[...] = jax.lax.dot_general(
        nz_ref[...], ctx_ref[...], (((1,), (1,)), ((), ())),
        preferred_element_type=jnp.float32)


def _recon(new_z, ctx_emb):
    return pl.pallas_call(
        _recon_body,
        grid=(4, 8),
        in_specs=[
            pl.BlockSpec((1024, D_VG), lambda i, j: (i, 0)),
            pl.BlockSpec((1280, D_VG), lambda i, j: (j, 0)),
        ],
        out_specs=pl.BlockSpec((1024, 1280), lambda i, j: (i, j)),
        out_shape=jax.ShapeDtypeStruct((N_TRAIN, N_NODES), jnp.float32),
    )(new_z, ctx_emb)


def _zq_body(out2_ref, cl_ref, res2_ref, z_ref, q_ref, qto_ref):
    o = jnp.concatenate([out2_ref[0], out2_ref[1]], axis=1)
    z = jnp.where(o > 0.0, o, jnp.exp(jnp.minimum(o, 0.0)) - 1.0)
    z_ref[...] = z
    cl = cl_ref[...]
    zc = jax.lax.dot_general(z, cl, (((1,), (1,)), ((), ())),
                             preferred_element_type=jnp.float32)
    z2 = jnp.sum(z * z, axis=1, keepdims=True)
    c2 = jnp.sum(cl * cl, axis=1)[None, :]
    d2 = z2 - 2.0 * zc + c2
    q = 1.0 / (1.0 + d2 / V)
    q = q / jnp.sum(q, axis=1, keepdims=True)
    q_ref[...] = q
    qto = q + 0.5 * (res2_ref[0, :, :K] + res2_ref[1, :, :K])
    qto_ref[...] = qto / jnp.sum(jnp.abs(qto), axis=1, keepdims=True)


def _zq(out2, cluster_layer, res2):
    return pl.pallas_call(
        _zq_body,
        grid=(10,),
        in_specs=[
            pl.BlockSpec((2, 1000, 128), lambda i: (0, i, 0)),
            pl.BlockSpec((K, D_EMB), lambda i: (0, 0)),
            pl.BlockSpec((2, 1000, 128), lambda i: (0, i, 0)),
        ],
        out_specs=[
            pl.BlockSpec((1000, D_EMB), lambda i: (i, 0)),
            pl.BlockSpec((1000, K), lambda i: (i, 0)),
            pl.BlockSpec((1000, K), lambda i: (i, 0)),
        ],
        out_shape=[
            jax.ShapeDtypeStruct((N_NODES, D_EMB), jnp.float32),
            jax.ShapeDtypeStruct((N_NODES, K), jnp.float32),
            jax.ShapeDtypeStruct((N_NODES, K), jnp.float32),
        ],
    )(out2, cluster_layer, res2)


def _reduce_rden(parts):
    return pl.pallas_call(
        _rden_body,
        grid=(5,),
        in_specs=[pl.BlockSpec((_NTILES, 2048), lambda i: (0, i))],
        out_specs=pl.BlockSpec((1, 2048), lambda i: (0, i)),
        out_shape=jax.ShapeDtypeStruct((1, N_NODES), jnp.float32),
    )(parts)[0]


def kernel(x, edge_index, w, c, temp, gumbel, W, att_src, att_dst,
           cluster_layer, node_emb, ctx_emb, comm_w):
    hh, a_src_p, a_dst_p, m2 = _project(x, W, att_src, att_dst)
    a_src = a_src_p[:N_NODES]
    a_dst = a_dst_p[:N_NODES]
    m = jax.nn.leaky_relu(m2[0, 0] + m2[0, 1], negative_slope=ALPHA)
    m16 = jnp.full((16,), m, jnp.float32)

    src = edge_index[0]
    dst = edge_index[1]
    ex, parts = _edge_logits(a_src, a_dst, src, dst, m16)
    rden = _reduce_rden(parts)
    me = jnp.stack(
        [src.reshape(16, _NCH, _G),
         lax.bitcast_convert_type(ex, jnp.int32).reshape(16, _NCH, _G)],
        axis=2,
    )
    (out2,) = _aggregate(me, dst.reshape(16, _NCH, _G), rden, hh)

    wc32 = jnp.stack([w.reshape(_NTILES, _PPT), c.reshape(_NTILES, _PPT)],
                     axis=1)
    ew, ewec = _pair_gather(wc32, node_emb)
    prior, q_vg, new_z = _vg_dense(ew, ewec, comm_w, gumbel, temp)
    recon_c = _recon(new_z, ctx_emb)

    wc16 = jnp.concatenate(
        [w.reshape(16, 2, 128), c.reshape(16, 2, 128)], axis=1)
    qvp = jnp.pad(q_vg, ((0, 0), (0, 128 - K)))
    (res2,) = _res_scatter(wc16, qvp.reshape(16, _RPT, 128))

    z, q, Q_to = _zq(out2, cluster_layer, res2)
    return (z, q, Q_to, prior, recon_c, q_vg, node_emb, comm_w)


# speedup vs baseline: 1.0862x; 1.0011x over previous
import jax
import jax.numpy as jnp
from jax import lax
from jax.experimental import pallas as pl
from jax.experimental.pallas import tpu as pltpu
from jax.experimental.pallas import tpu_sc as plsc

N_NODES = 10000
N_EDGES = 160000
D_FEAT = 256
D_EMB = 256
D_VG = 128
K = 16
N_TRAIN = 4096
V = 1.0
ALPHA = 0.2

_NPAD = 10240


def _proj_body(x_ref, W_ref, asrc_ref, adst_ref, hh_ref, a2_ref, m2_ref):
    i = pl.program_id(0)
    h = jnp.dot(x_ref[...], W_ref[...], preferred_element_type=jnp.float32)
    hh_ref[0] = h[:, :128]
    hh_ref[1] = h[:, 128:]
    a2 = jnp.concatenate(
        [h @ asrc_ref[...].reshape(D_EMB, 1), h @ adst_ref[...].reshape(D_EMB, 1)],
        axis=1,
    )
    a2_ref[...] = a2
    rows = i * 1024 + lax.broadcasted_iota(jnp.int32, (1024, 2), 0)
    a2m = jnp.where(rows < N_NODES, a2, -1e30)
    bmax = jnp.max(a2m, axis=0, keepdims=True)

    @pl.when(i == 0)
    def _():
        m2_ref[...] = bmax

    @pl.when(i > 0)
    def _():
        m2_ref[...] = jnp.maximum(m2_ref[...], bmax)


def _project(x, W, att_src, att_dst):
    xp = jnp.zeros((_NPAD, D_FEAT), jnp.float32).at[:N_NODES].set(x)
    grid = _NPAD // 1024
    hh, a2, m2 = pl.pallas_call(
        _proj_body,
        grid=(grid,),
        in_specs=[
            pl.BlockSpec((1024, D_FEAT), lambda i: (i, 0)),
            pl.BlockSpec((D_FEAT, D_EMB), lambda i: (0, 0)),
            pl.BlockSpec((D_EMB,), lambda i: (0,)),
            pl.BlockSpec((D_EMB,), lambda i: (0,)),
        ],
        out_specs=[
            pl.BlockSpec((2, 1024, 128), lambda i: (0, i, 0)),
            pl.BlockSpec((1024, 2), lambda i: (i, 0)),
            pl.BlockSpec((1, 2), lambda i: (0, 0)),
        ],
        out_shape=[
            jax.ShapeDtypeStruct((2, _NPAD, 128), jnp.float32),
            jax.ShapeDtypeStruct((_NPAD, 2), jnp.float32),
            jax.ShapeDtypeStruct((1, 2), jnp.float32),
        ],
    )(xp, W, att_src, att_dst)
    return hh, a2[:, 0], a2[:, 1], m2


_NTILES = 32
_EPT = N_EDGES // _NTILES
_EPT_PAD = ((_EPT + 15) // 16) * 16
_FULL_CHUNKS = _EPT // 16
_TAIL = _EPT - _FULL_CHUNKS * 16

_SC_MESH = plsc.VectorSubcoreMesh(core_axis_name="c", subcore_axis_name="s")
_SC_PARAMS = pltpu.CompilerParams(needs_layout_passes=False)


def _edge_logits_body(asrc_hbm, adst_hbm, src_hbm, dst_hbm, m_hbm,
                      ex_hbm, part_hbm,
                      asrc_v, adst_v, srcv, dstv, exv, denv, m_v):
    cid = lax.axis_index("c")
    sid = lax.axis_index("s")
    wid = sid * 2 + cid
    ebase = wid * _EPT

    pltpu.sync_copy(asrc_hbm, asrc_v)
    pltpu.sync_copy(adst_hbm, adst_v)
    pltpu.sync_copy(src_hbm.at[pl.ds(ebase, _EPT)], srcv.at[pl.ds(0, _EPT)])
    pltpu.sync_copy(dst_hbm.at[pl.ds(ebase, _EPT)], dstv.at[pl.ds(0, _EPT)])
    pltpu.sync_copy(m_hbm, m_v)
    m = m_v[...]

    zero16 = jnp.zeros((16,), jnp.float32)

    def _zero_body(i, carry):
        denv[pl.ds(i * 16, 16)] = zero16
        return carry

    lax.fori_loop(0, N_NODES // 16, _zero_body, 0)

    lanes = lax.iota(jnp.int32, 16)

    def _chunk(i, mask):
        off = i * 16
        sv = srcv[pl.ds(off, 16)]
        dv = dstv[pl.ds(off, 16)]
        if mask is not None:
            sv = jnp.where(mask, sv, 0)
            dv = jnp.where(mask, dv, 0)
        sa = plsc.load_gather(asrc_v, [sv])
        da = plsc.load_gather(adst_v, [dv])
        e = sa + da
        e = jnp.where(e >= 0.0, e, ALPHA * e)
        ex = jnp.exp(e - m)
        exv[pl.ds(off, 16)] = ex
        if mask is None:
            plsc.addupdate_scatter(denv, [dv], ex)
        else:
            plsc.addupdate_scatter(denv, [dv], ex, mask=mask)

    def _loop_body(i, carry):
        _chunk(i, None)
        return carry

    lax.fori_loop(0, _FULL_CHUNKS, _loop_body, 0)
    _chunk(_FULL_CHUNKS, lanes < _TAIL)

    pltpu.sync_copy(exv.at[pl.ds(0, _EPT)], ex_hbm.at[pl.ds(ebase, _EPT)])
    pltpu.sync_copy(denv, part_hbm.at[wid])


_edge_logits = pl.kernel(
    _edge_logits_body,
    out_type=[
        jax.ShapeDtypeStruct((N_EDGES,), jnp.float32),
        jax.ShapeDtypeStruct((_NTILES, N_NODES), jnp.float32),
    ],
    mesh=_SC_MESH,
    compiler_params=_SC_PARAMS,
    scratch_types=[
        pltpu.VMEM((N_NODES,), jnp.float32),
        pltpu.VMEM((N_NODES,), jnp.float32),
        pltpu.VMEM((_EPT_PAD,), jnp.int32),
        pltpu.VMEM((_EPT_PAD,), jnp.int32),
        pltpu.VMEM((_EPT_PAD,), jnp.float32),
        pltpu.VMEM((N_NODES,), jnp.float32),
        pltpu.VMEM((16,), jnp.float32),
    ],
)


_G = 80
_EPS = N_EDGES // 16
_NCH = _EPS // _G
_ROWS_T = N_NODES // 16


_STRIPE = 632
_STRIPE_L = N_NODES - 15 * _STRIPE


def _agg_body(me_hbm, dst2_hbm, rden_hbm, hh_hbm,
              out_hbm,
              rden_v, dst2_v, mbA, attn_v, gb0, acc_sh, gsem0):
    cid = lax.axis_index("c")
    sid = lax.axis_index("s")

    pltpu.sync_copy(rden_hbm, rden_v)
    pltpu.sync_copy(dst2_hbm.at[sid], dst2_v)

    zero16 = jnp.zeros((16,), jnp.float32)

    def _zrow(i, carry):
        for r in range(8):
            gb0[i, pl.ds(r * 16, 16)] = zero16
        return carry

    lax.fori_loop(0, _G, _zrow, 0)
    base = sid * _STRIPE

    @pl.when(sid < 15)
    def _():
        for k in range(7):
            pltpu.sync_copy(gb0, acc_sh.at[pl.ds(base + k * _G, _G)])
        pltpu.sync_copy(gb0.at[pl.ds(0, _STRIPE - 7 * _G)],
                        acc_sh.at[pl.ds(base + 7 * _G, _STRIPE - 7 * _G)])

    @pl.when(sid == 15)
    def _():
        for k in range(6):
            pltpu.sync_copy(gb0, acc_sh.at[pl.ds(base + k * _G, _G)])
        pltpu.sync_copy(gb0.at[pl.ds(0, _STRIPE_L - 6 * _G)],
                        acc_sh.at[pl.ds(base + 6 * _G, _STRIPE_L - 6 * _G)])

    plsc.subcore_barrier()

    def _chunk(ci, carry):
        pltpu.sync_copy(me_hbm.at[sid, ci], mbA)
        pltpu.async_copy(hh_hbm.at[cid].at[mbA.at[0]], gb0, gsem0).wait()
        for g in range(_G // 16):
            dv = dst2_v[ci, pl.ds(g * 16, 16)]
            rv = plsc.load_gather(rden_v, [dv])
            exv = plsc.bitcast(mbA[1, pl.ds(g * 16, 16)], jnp.float32)
            attn_v[pl.ds(g * 16, 16)] = exv * rv

        def _egroup(g, c2):
            avec = attn_v[pl.ds(g * 16, 16)]
            e0 = g * 16
            for j in range(16):
                av = jnp.broadcast_to(avec[j], (16,))
                for r in range(8):
                    gb0[e0 + j, pl.ds(r * 16, 16)] = (
                        gb0[e0 + j, pl.ds(r * 16, 16)] * av)
            return c2

        lax.fori_loop(0, _G // 16, _egroup, 0)
        pltpu.sync_copy(gb0, acc_sh.at[dst2_v.at[ci]], add=True)
        return carry

    lax.fori_loop(0, _NCH, _chunk, 0)
    plsc.subcore_barrier()

    @pl.when(sid < 15)
    def _():
        pltpu.sync_copy(acc_sh.at[pl.ds(base, _STRIPE)],
                        out_hbm.at[cid, pl.ds(base, _STRIPE)])

    @pl.when(sid == 15)
    def _():
        pltpu.sync_copy(acc_sh.at[pl.ds(base, _STRIPE_L)],
                        out_hbm.at[cid, pl.ds(base, _STRIPE_L)])


_aggregate = pl.kernel(
    _agg_body,
    out_type=[
        jax.ShapeDtypeStruct((2, N_NODES, 128), jnp.float32),
    ],
    mesh=_SC_MESH,
    compiler_params=_SC_PARAMS,
    scratch_types=[
        pltpu.VMEM((N_NODES,), jnp.float32),
        pltpu.VMEM((_NCH, _G), jnp.int32),
        pltpu.VMEM((2, _G), jnp.int32),
        pltpu.VMEM((_G,), jnp.float32),
        pltpu.VMEM((_G, 128), jnp.float32),
        pltpu.VMEM_SHARED((N_NODES, 128), jnp.float32),
        pltpu.SemaphoreType.DMA,
    ],
)


def _rden_body(part_ref, out_ref):
    s = jnp.sum(part_ref[...], axis=0, keepdims=True)
    out_ref[...] = 1.0 / (s + 1e-16)


_PPT = N_TRAIN // _NTILES


def _pair_gather_body(wc_hbm, emb_hbm, ew_hbm, ewec_hbm,
                      wc_v, ew_v, ec_v, sem):
    cid = lax.axis_index("c")
    sid = lax.axis_index("s")
    wid = sid * 2 + cid
    base = wid * _PPT

    pltpu.sync_copy(wc_hbm.at[wid], wc_v)
    pltpu.async_copy(emb_hbm.at[wc_v.at[0]], ew_v, sem).wait()
    pltpu.async_copy(emb_hbm.at[wc_v.at[1]], ec_v, sem).wait()

    def _mul(i, carry):
        for r in range(8):
            ec_v[i, pl.ds(r * 16, 16)] = (
                ec_v[i, pl.ds(r * 16, 16)] * ew_v[i, pl.ds(r * 16, 16)])
        return carry

    lax.fori_loop(0, _PPT, _mul, 0)
    pltpu.sync_copy(ew_v, ew_hbm.at[pl.ds(base, _PPT)])
    pltpu.sync_copy(ec_v, ewec_hbm.at[pl.ds(base, _PPT)])


_pair_gather = pl.kernel(
    _pair_gather_body,
    out_type=[
        jax.ShapeDtypeStruct((N_TRAIN, D_VG), jnp.float32),
        jax.ShapeDtypeStruct((N_TRAIN, D_VG), jnp.float32),
    ],
    mesh=_SC_MESH,
    compiler_params=_SC_PARAMS,
    scratch_types=[
        pltpu.VMEM((2, _PPT), jnp.int32),
        pltpu.VMEM((_PPT, D_VG), jnp.float32),
        pltpu.VMEM((_PPT, D_VG), jnp.float32),
        pltpu.SemaphoreType.DMA,
    ],
)


_RPT = N_TRAIN // 16


def _res_body(wc_hbm, qv_hbm, res_hbm,
              wc_v, qv_v, zbuf, acc_sh):
    cid = lax.axis_index("c")
    sid = lax.axis_index("s")

    pltpu.sync_copy(wc_hbm.at[sid], wc_v)

    zero16 = jnp.zeros((16,), jnp.float32)

    def _z(i, carry):
        for r in range(8):
            zbuf[i, pl.ds(r * 16, 16)] = zero16
        return carry

    lax.fori_loop(0, 104, _z, 0)
    base = sid * _STRIPE

    @pl.when(sid < 15)
    def _():
        for k in range(6):
            pltpu.sync_copy(zbuf, acc_sh.at[pl.ds(base + k * 104, 104)])
        pltpu.sync_copy(zbuf.at[pl.ds(0, 8)],
                        acc_sh.at[pl.ds(base + 624, 8)])

    @pl.when(sid == 15)
    def _():
        for k in range(5):
            pltpu.sync_copy(zbuf, acc_sh.at[pl.ds(base + k * 104, 104)])

    plsc.subcore_barrier()
    for k in range(_RPT // 128):
        pltpu.sync_copy(qv_hbm.at[sid, pl.ds(k * 128, 128)], qv_v)
        pltpu.sync_copy(qv_v,
                        acc_sh.at[wc_v.at[cid * (_RPT // 128) + k]],
                        add=True)
    plsc.subcore_barrier()

    @pl.when(sid < 15)
    def _():
        pltpu.sync_copy(acc_sh.at[pl.ds(base, _STRIPE)],
                        res_hbm.at[cid, pl.ds(base, _STRIPE)])

    @pl.when(sid == 15)
    def _():
        pltpu.sync_copy(acc_sh.at[pl.ds(base, _STRIPE_L)],
                        res_hbm.at[cid, pl.ds(base, _STRIPE_L)])


_res_scatter = pl.kernel(
    _res_body,
    out_type=[jax.ShapeDtypeStruct((2, N_NODES, 128), jnp.float32)],
    mesh=_SC_MESH,
    compiler_params=_SC_PARAMS,
    scratch_types=[
        pltpu.VMEM((4, 128), jnp.int32),
        pltpu.VMEM((128, 128), jnp.float32),
        pltpu.VMEM((104, 128), jnp.float32),
        pltpu.VMEM_SHARED((N_NODES, 128), jnp.float32),
    ],
)


def _vg_dense_body(ew_ref, ewec_ref, cw_ref, gum_ref, tmp_ref,
                   prior_ref, qvg_ref, nz_ref):
    cw = cw_ref[...]
    dn = (((1,), (1,)), ((), ()))
    pl_ = jax.lax.dot_general(ew_ref[...], cw, dn,
                              preferred_element_type=jnp.float32)
    ql = jax.lax.dot_general(ewec_ref[...], cw, dn,
                             preferred_element_type=jnp.float32)

    def _softmax(x):
        mx = jnp.max(x, axis=1, keepdims=True)
        e = jnp.exp(x - mx)
        return e / jnp.sum(e, axis=1, keepdims=True)

    prior_ref[...] = _softmax(pl_)
    qvg_ref[...] = _softmax(ql)
    zg = _softmax((ql + gum_ref[...]) / tmp_ref[0, 0])
    nz_ref[...] = jnp.dot(zg, cw, preferred_element_type=jnp.float32)


def _vg_dense(ew, ewec, comm_w, gumbel, temp):
    return pl.pallas_call(
        _vg_dense_body,
        grid=(4,),
        in_specs=[
            pl.BlockSpec((1024, D_VG), lambda i: (i, 0)),
            pl.BlockSpec((1024, D_VG), lambda i: (i, 0)),
            pl.BlockSpec((K, D_VG), lambda i: (0, 0)),
            pl.BlockSpec((1024, K), lambda i: (i, 0)),
            pl.BlockSpec((1, 1), lambda i: (0, 0)),
        ],
        out_specs=[
            pl.BlockSpec((1024, K), lambda i: (i, 0)),
            pl.BlockSpec((1024, K), lambda i: (i, 0)),
            pl.BlockSpec((1024, D_VG), lambda i: (i, 0)),
        ],
        out_shape=[
            jax.ShapeDtypeStruct((N_TRAIN, K), jnp.float32),
            jax.ShapeDtypeStruct((N_TRAIN, K), jnp.float32),
            jax.ShapeDtypeStruct((N_TRAIN, D_VG), jnp.float32),
        ],
    )(ew, ewec, comm_w, gumbel, temp.reshape(1, 1))


def _recon_body(nz_ref, ctx_ref, out_ref):
    out_ref[...] = jax.lax.dot_general(
        nz_ref[...], ctx_ref[...], (((1,), (1,)), ((), ())),
        preferred_element_type=jnp.float32)


def _recon(new_z, ctx_emb):
    return pl.pallas_call(
        _recon_body,
        grid=(4, 8),
        in_specs=[
            pl.BlockSpec((1024, D_VG), lambda i, j: (i, 0)),
            pl.BlockSpec((1280, D_VG), lambda i, j: (j, 0)),
        ],
        out_specs=pl.BlockSpec((1024, 1280), lambda i, j: (i, j)),
        out_shape=jax.ShapeDtypeStruct((N_TRAIN, N_NODES), jnp.float32),
    )(new_z, ctx_emb)


def _zq_body(out2_ref, cl_ref, res2_ref, z_ref, q_ref, qto_ref):
    o = jnp.concatenate([out2_ref[0], out2_ref[1]], axis=1)
    z = jnp.where(o > 0.0, o, jnp.exp(jnp.minimum(o, 0.0)) - 1.0)
    z_ref[...] = z
    cl = cl_ref[...]
    zc = jax.lax.dot_general(z, cl, (((1,), (1,)), ((), ())),
                             preferred_element_type=jnp.float32)
    z2 = jnp.sum(z * z, axis=1, keepdims=True)
    c2 = jnp.sum(cl * cl, axis=1)[None, :]
    d2 = z2 - 2.0 * zc + c2
    q = 1.0 / (1.0 + d2 / V)
    q = q / jnp.sum(q, axis=1, keepdims=True)
    q_ref[...] = q
    qto = q + 0.5 * (res2_ref[0, :, :K] + res2_ref[1, :, :K])
    qto_ref[...] = qto / jnp.sum(jnp.abs(qto), axis=1, keepdims=True)


def _zq(out2, cluster_layer, res2):
    return pl.pallas_call(
        _zq_body,
        grid=(10,),
        in_specs=[
            pl.BlockSpec((2, 1000, 128), lambda i: (0, i, 0)),
            pl.BlockSpec((K, D_EMB), lambda i: (0, 0)),
            pl.BlockSpec((2, 1000, 128), lambda i: (0, i, 0)),
        ],
        out_specs=[
            pl.BlockSpec((1000, D_EMB), lambda i: (i, 0)),
            pl.BlockSpec((1000, K), lambda i: (i, 0)),
            pl.BlockSpec((1000, K), lambda i: (i, 0)),
        ],
        out_shape=[
            jax.ShapeDtypeStruct((N_NODES, D_EMB), jnp.float32),
            jax.ShapeDtypeStruct((N_NODES, K), jnp.float32),
            jax.ShapeDtypeStruct((N_NODES, K), jnp.float32),
        ],
    )(out2, cluster_layer, res2)


def _reduce_rden(parts):
    return pl.pallas_call(
        _rden_body,
        grid=(5,),
        in_specs=[pl.BlockSpec((_NTILES, 2048), lambda i: (0, i))],
        out_specs=pl.BlockSpec((1, 2048), lambda i: (0, i)),
        out_shape=jax.ShapeDtypeStruct((1, N_NODES), jnp.float32),
    )(parts)[0]


def kernel(x, edge_index, w, c, temp, gumbel, W, att_src, att_dst,
           cluster_layer, node_emb, ctx_emb, comm_w):
    hh, a_src_p, a_dst_p, m2 = _project(x, W, att_src, att_dst)
    a_src = a_src_p[:N_NODES]
    a_dst = a_dst_p[:N_NODES]
    m = jax.nn.leaky_relu(m2[0, 0] + m2[0, 1], negative_slope=ALPHA)
    m16 = jnp.full((16,), m, jnp.float32)

    src = edge_index[0]
    dst = edge_index[1]
    ex, parts = _edge_logits(a_src, a_dst, src, dst, m16)
    rden = _reduce_rden(parts)
    me = jnp.stack(
        [src.reshape(16, _NCH, _G),
         lax.bitcast_convert_type(ex, jnp.int32).reshape(16, _NCH, _G)],
        axis=2,
    )
    (out2,) = _aggregate(me, dst.reshape(16, _NCH, _G), rden, hh)

    wc32 = jnp.stack([w.reshape(_NTILES, _PPT), c.reshape(_NTILES, _PPT)],
                     axis=1)
    ew, ewec = _pair_gather(wc32, node_emb)
    prior, q_vg, new_z = _vg_dense(ew, ewec, comm_w, gumbel, temp)
    recon_c = _recon(new_z, ctx_emb)

    wc16 = jnp.concatenate(
        [w.reshape(16, 2, 128), c.reshape(16, 2, 128)], axis=1)
    qvp = jnp.pad(q_vg, ((0, 0), (0, 128 - K)))
    (res2,) = _res_scatter(wc16, qvp.reshape(16, _RPT, 128))

    z, q, Q_to = _zq(out2, cluster_layer, res2)
    return (z, q, Q_to, prior, recon_c, q_vg, node_emb, comm_w)
